# baseline (speedup 1.0000x reference)
import jax
import jax.numpy as jnp
from jax import lax
from jax.experimental import pallas as pl
from jax.experimental.pallas import tpu as pltpu
from jax.experimental.pallas import tpu_sc as plsc

B = 16384
NF = 26
V = 100000
D = 16
NCONT = 13
H = 16
OUT = 16
EPS = 1e-5

SC_CORES = 2
SC_SUBCORES = 16
NW = SC_CORES * SC_SUBCORES
TOTAL = B * NF
CH = B // NW
IW = 128
NG = CH // IW

PACK = 8
R = B // PACK
BLK_R = 256

VB = 8192
NVB = -(-V // VB)


def _tr_body(in_ref, out_ref):
    x = in_ref[0].reshape(D, VB // 8, 8)
    tr = jnp.transpose(x, (1, 2, 0))
    out_ref[0] = tr.reshape(VB // 8, 8 * D)


def _transpose(tables_t):
    return pl.pallas_call(
        _tr_body,
        grid=(NF, NVB),
        in_specs=[pl.BlockSpec((1, D, VB), lambda f, j: (f, 0, j))],
        out_specs=pl.BlockSpec((1, VB // 8, 8 * D), lambda f, j: (f, j, 0)),
        out_shape=jax.ShapeDtypeStruct((NF, V // 8, 8 * D), jnp.float32),
    )(tables_t)


def _gather_body(tbl_hbm, idx_hbm, out_hbm, idx_v, rows_v, sem):
    c = lax.axis_index("c")
    s = lax.axis_index("s")
    wid = s * SC_CORES + c
    bw = wid * CH

    def step(f, carry):
        base = f * B + bw
        for j in range(NG):
            pltpu.sync_copy(idx_hbm.at[pl.ds(base + j * IW, IW)], idx_v.at[j])
        off = f * V
        for j in range(NG):
            for k in range(IW // 16):
                idx_v[j, pl.ds(k * 16, 16)] = idx_v[j, pl.ds(k * 16, 16)] + off
        cps = [
            pltpu.async_copy(tbl_hbm.at[idx_v.at[j]],
                             rows_v.at[pl.ds(j * IW, IW)], sem)
            for j in range(NG)
        ]
        for cp in cps:
            cp.wait()
        pltpu.sync_copy(rows_v, out_hbm.at[pl.ds(base, CH)])
        return carry

    lax.fori_loop(0, NF, step, 0)


def _sc_gather(tbl_rm, idx_fm):
    mesh = plsc.VectorSubcoreMesh(core_axis_name="c", subcore_axis_name="s")
    return pl.kernel(
        _gather_body,
        mesh=mesh,
        compiler_params=pltpu.CompilerParams(use_tc_tiling_on_sc=False),
        out_type=jax.ShapeDtypeStruct((TOTAL, D), jnp.float32),
        scratch_types=[
            pltpu.VMEM((NG, IW), jnp.int32),
            pltpu.VMEM((CH, D), jnp.float32),
            pltpu.SemaphoreType.DMA,
        ],
    )(tbl_rm, idx_fm)


def _mlp_body(emb_ref, xc_ref, k1e_ref, k1c_ref, b1_ref, k2_ref, b2_ref,
              k3_ref, b3_ref, k4_ref, b4_ref, g_ref, be_ref, mu_ref, var_ref,
              out_ref):
    f32 = jnp.float32
    hi = lax.Precision.HIGHEST
    dot = lambda a, b: jnp.dot(a, b, preferred_element_type=f32, precision=hi)
    xc = xc_ref[...]
    x2 = (xc - mu_ref[...]) * (g_ref[...] * lax.rsqrt(var_ref[...] + EPS)) \
        + be_ref[...]
    h = dot(x2, k1c_ref[...])
    for f in range(NF):
        h = h + dot(emb_ref[f], k1e_ref[f])
    h = jnp.maximum(h + b1_ref[...], 0.0)
    h = jnp.maximum(dot(h, k2_ref[...]) + b2_ref[...], 0.0)
    h = jnp.maximum(dot(h, k3_ref[...]) + b3_ref[...], 0.0)
    out_ref[...] = dot(h, k4_ref[...]) + b4_ref[...]


def _mlp(emb_p, xc_p, k1e, k1c, b1p, k2, b2p, k3, b3p, k4, b4p,
         gp, bep, mup, varp):
    full2 = lambda shape: pl.BlockSpec(shape, lambda i: (0, 0))
    full3 = lambda shape: pl.BlockSpec(shape, lambda i: (0, 0, 0))
    return pl.pallas_call(
        _mlp_body,
        grid=(R // BLK_R,),
        in_specs=[
            pl.BlockSpec((NF, BLK_R, PACK * D), lambda i: (0, i, 0)),
            pl.BlockSpec((BLK_R, PACK * NCONT), lambda i: (i, 0)),
            full3((NF, PACK * D, PACK * H)),
            full2((PACK * NCONT, PACK * H)),
            full2((1, PACK * H)),
            full2((PACK * H, PACK * H)),
            full2((1, PACK * H)),
            full2((PACK * H, PACK * H)),
            full2((1, PACK * H)),
            full2((PACK * H, PACK * OUT)),
            full2((1, PACK * OUT)),
            full2((1, PACK * NCONT)),
            full2((1, PACK * NCONT)),
            full2((1, PACK * NCONT)),
            full2((1, PACK * NCONT)),
        ],
        out_specs=pl.BlockSpec((BLK_R, PACK * OUT), lambda i: (i, 0)),
        out_shape=jax.ShapeDtypeStruct((R, PACK * OUT), jnp.float32),
    )(emb_p, xc_p, k1e, k1c, b1p, k2, b2p, k3, b3p, k4, b4p, gp, bep, mup,
      varp)


def kernel(x_cont, x_cat, tables, W1, b1, W2, b2, W3, b3, W4, b4,
           bn_gamma, bn_beta, bn_mean, bn_var):
    f32 = jnp.float32
    idx_fm = x_cat.T.reshape(TOTAL)
    tbl_rm = tables.reshape(NF * V, D)
    emb = _sc_gather(tbl_rm, idx_fm)
    emb_p = emb.reshape(NF, R, PACK * D)

    eye8 = jnp.eye(PACK, dtype=f32)
    kron8 = lambda m: jnp.kron(eye8, m)
    m1e = W1[:, :NF * D].T.reshape(NF, D, H)
    k1e = jax.vmap(kron8)(m1e)
    k1c = kron8(W1[:, NF * D:].T)
    k2 = kron8(W2.T)
    k3 = kron8(W3.T)
    k4 = kron8(W4.T)
    tile8 = lambda v: jnp.tile(v.reshape(1, -1), (1, PACK))
    xc_p = x_cont.reshape(R, PACK * NCONT)

    out_p = _mlp(emb_p, xc_p, k1e, k1c, tile8(b1), k2, tile8(b2), k3,
                 tile8(b3), k4, tile8(b4), tile8(bn_gamma), tile8(bn_beta),
                 tile8(bn_mean), tile8(bn_var))
    return out_p.reshape(B, OUT)


# speedup vs baseline: 1.0013x; 1.0013x over previous
import jax
import jax.numpy as jnp
from jax import lax
from jax.experimental import pallas as pl
from jax.experimental.pallas import tpu as pltpu
from jax.experimental.pallas import tpu_sc as plsc

B = 16384
NF = 26
V = 100000
D = 16
NCONT = 13
H = 16
OUT = 16
EPS = 1e-5

SC_CORES = 2
SC_SUBCORES = 16
NW = SC_CORES * SC_SUBCORES
TOTAL = B * NF
CH = B // NW
IW = 128
NG = CH // IW

PACK = 8
R = B // PACK
BLK_R = 512

VB = 8192
NVB = -(-V // VB)


def _tr_body(in_ref, out_ref):
    x = in_ref[0].reshape(D, VB // 8, 8)
    tr = jnp.transpose(x, (1, 2, 0))
    out_ref[0] = tr.reshape(VB // 8, 8 * D)


def _transpose(tables_t):
    return pl.pallas_call(
        _tr_body,
        grid=(NF, NVB),
        in_specs=[pl.BlockSpec((1, D, VB), lambda f, j: (f, 0, j))],
        out_specs=pl.BlockSpec((1, VB // 8, 8 * D), lambda f, j: (f, j, 0)),
        out_shape=jax.ShapeDtypeStruct((NF, V // 8, 8 * D), jnp.float32),
    )(tables_t)


def _gather_body(tbl_hbm, idx_hbm, out_hbm, idx_v, rows_v, sem):
    c = lax.axis_index("c")
    s = lax.axis_index("s")
    wid = s * SC_CORES + c
    bw = wid * CH

    def step(f, carry):
        base = f * B + bw
        for j in range(NG):
            pltpu.sync_copy(idx_hbm.at[pl.ds(base + j * IW, IW)], idx_v.at[j])
        off = f * V
        for j in range(NG):
            for k in range(IW // 16):
                idx_v[j, pl.ds(k * 16, 16)] = idx_v[j, pl.ds(k * 16, 16)] + off
        cps = [
            pltpu.async_copy(tbl_hbm.at[idx_v.at[j]],
                             rows_v.at[pl.ds(j * IW, IW)], sem)
            for j in range(NG)
        ]
        for cp in cps:
            cp.wait()
        pltpu.sync_copy(rows_v, out_hbm.at[pl.ds(base, CH)])
        return carry

    lax.fori_loop(0, NF, step, 0)


def _sc_gather(tbl_rm, idx_fm):
    mesh = plsc.VectorSubcoreMesh(core_axis_name="c", subcore_axis_name="s")
    return pl.kernel(
        _gather_body,
        mesh=mesh,
        compiler_params=pltpu.CompilerParams(use_tc_tiling_on_sc=False),
        out_type=jax.ShapeDtypeStruct((TOTAL, D), jnp.float32),
        scratch_types=[
            pltpu.VMEM((NG, IW), jnp.int32),
            pltpu.VMEM((CH, D), jnp.float32),
            pltpu.SemaphoreType.DMA,
        ],
    )(tbl_rm, idx_fm)


def _mlp_body(emb_ref, xc_ref, k1e_ref, k1c_ref, b1_ref, k2_ref, b2_ref,
              k3_ref, b3_ref, k4_ref, b4_ref, g_ref, be_ref, mu_ref, var_ref,
              out_ref):
    f32 = jnp.float32
    hi = lax.Precision.HIGHEST
    dot = lambda a, b: jnp.dot(a, b, preferred_element_type=f32, precision=hi)
    xc = xc_ref[...]
    x2 = (xc - mu_ref[...]) * (g_ref[...] * lax.rsqrt(var_ref[...] + EPS)) \
        + be_ref[...]
    h = dot(x2, k1c_ref[...])
    for f in range(NF):
        h = h + dot(emb_ref[f], k1e_ref[f])
    h = jnp.maximum(h + b1_ref[...], 0.0)
    h = jnp.maximum(dot(h, k2_ref[...]) + b2_ref[...], 0.0)
    h = jnp.maximum(dot(h, k3_ref[...]) + b3_ref[...], 0.0)
    out_ref[...] = dot(h, k4_ref[...]) + b4_ref[...]


def _mlp(emb_p, xc_p, k1e, k1c, b1p, k2, b2p, k3, b3p, k4, b4p,
         gp, bep, mup, varp):
    full2 = lambda shape: pl.BlockSpec(shape, lambda i: (0, 0))
    full3 = lambda shape: pl.BlockSpec(shape, lambda i: (0, 0, 0))
    return pl.pallas_call(
        _mlp_body,
        grid=(R // BLK_R,),
        in_specs=[
            pl.BlockSpec((NF, BLK_R, PACK * D), lambda i: (0, i, 0)),
            pl.BlockSpec((BLK_R, PACK * NCONT), lambda i: (i, 0)),
            full3((NF, PACK * D, PACK * H)),
            full2((PACK * NCONT, PACK * H)),
            full2((1, PACK * H)),
            full2((PACK * H, PACK * H)),
            full2((1, PACK * H)),
            full2((PACK * H, PACK * H)),
            full2((1, PACK * H)),
            full2((PACK * H, PACK * OUT)),
            full2((1, PACK * OUT)),
            full2((1, PACK * NCONT)),
            full2((1, PACK * NCONT)),
            full2((1, PACK * NCONT)),
            full2((1, PACK * NCONT)),
        ],
        out_specs=pl.BlockSpec((BLK_R, PACK * OUT), lambda i: (i, 0)),
        out_shape=jax.ShapeDtypeStruct((R, PACK * OUT), jnp.float32),
    )(emb_p, xc_p, k1e, k1c, b1p, k2, b2p, k3, b3p, k4, b4p, gp, bep, mup,
      varp)


def kernel(x_cont, x_cat, tables, W1, b1, W2, b2, W3, b3, W4, b4,
           bn_gamma, bn_beta, bn_mean, bn_var):
    f32 = jnp.float32
    idx_fm = x_cat.T.reshape(TOTAL)
    tbl_rm = tables.reshape(NF * V, D)
    emb = _sc_gather(tbl_rm, idx_fm)
    emb_p = emb.reshape(NF, R, PACK * D)

    eye8 = jnp.eye(PACK, dtype=f32)
    kron8 = lambda m: jnp.kron(eye8, m)
    m1e = W1[:, :NF * D].T.reshape(NF, D, H)
    k1e = jax.vmap(kron8)(m1e)
    k1c = kron8(W1[:, NF * D:].T)
    k2 = kron8(W2.T)
    k3 = kron8(W3.T)
    k4 = kron8(W4.T)
    tile8 = lambda v: jnp.tile(v.reshape(1, -1), (1, PACK))
    xc_p = x_cont.reshape(R, PACK * NCONT)

    out_p = _mlp(emb_p, xc_p, k1e, k1c, tile8(b1), k2, tile8(b2), k3,
                 tile8(b3), k4, tile8(b4), tile8(bn_gamma), tile8(bn_beta),
                 tile8(bn_mean), tile8(bn_var))
    return out_p.reshape(B, OUT)
